# baseline (speedup 1.0000x reference)
import numpy as np
import jax
import jax.numpy as jnp
from jax.experimental import pallas as pl
from jax.experimental.pallas import tpu as pltpu

_C, _T, _H, _W = 3, 32, 384, 384
_ALPHA = 4
_NSLOW = _T // _ALPHA
_IDX = tuple(int(v) for v in np.linspace(0.0, _T - 1, _NSLOW).astype(np.float32))
_QUAD = 4
_NQ = _T // _QUAD
_NBUF = 8
assert all(_IDX[q] // _QUAD == q for q in range(_NQ))


def _body(in_ref, slow_ref, fast_ref, *scratch):
    bufs = scratch[:_NBUF]
    in_sem, fast_sem, slow_sem = scratch[_NBUF:]

    def in_cp(q):
        return pltpu.make_async_copy(
            in_ref.at[:, pl.ds(q * _QUAD, _QUAD)], bufs[q % _NBUF],
            in_sem.at[q % _NBUF])

    def fast_cp(q):
        return pltpu.make_async_copy(
            bufs[q % _NBUF], fast_ref.at[:, pl.ds(q * _QUAD, _QUAD)],
            fast_sem.at[q % _NBUF])

    def slow_cp(q):
        off = _IDX[q] - q * _QUAD
        return pltpu.make_async_copy(
            bufs[q % _NBUF].at[:, pl.ds(off, 1)], slow_ref.at[:, pl.ds(q, 1)],
            slow_sem.at[q % _NBUF])

    for q in range(_NQ):
        in_cp(q).start()
    for q in range(_NQ):
        in_cp(q).wait()
        fast_cp(q).start()
        slow_cp(q).start()
    for q in range(_NQ):
        fast_cp(q).wait()
        slow_cp(q).wait()


def kernel(frames):
    slow, fast = pl.pallas_call(
        _body,
        in_specs=[pl.BlockSpec(memory_space=pl.ANY)],
        out_specs=[
            pl.BlockSpec(memory_space=pl.ANY),
            pl.BlockSpec(memory_space=pl.ANY),
        ],
        out_shape=[
            jax.ShapeDtypeStruct((_C, _NSLOW, _H, _W), frames.dtype),
            jax.ShapeDtypeStruct((_C, _T, _H, _W), frames.dtype),
        ],
        scratch_shapes=(
            [pltpu.VMEM((_C, _QUAD, _H, _W), jnp.float32) for _ in range(_NBUF)]
            + [pltpu.SemaphoreType.DMA((_NBUF,)) for _ in range(3)]
        ),
        compiler_params=pltpu.CompilerParams(
            vmem_limit_bytes=100 * 1024 * 1024,
        ),
    )(frames)
    return (slow, fast)
